# Initial kernel scaffold; baseline (speedup 1.0000x reference)
import jax
import jax.numpy as jnp
from jax.experimental import pallas as pl


def kernel(x, x_lens, y_padded, y_lens, W_enc, b_enc, embed, W_encp, b_encp, W_decp, b_decp, W_out, b_out):
    raise NotImplementedError("write your pallas kernel here")


# speedup vs baseline: 3.5119x; 3.5119x over previous
import functools

import jax
import jax.numpy as jnp
from jax import lax
from jax.experimental import pallas as pl

B, T, F = 4, 512, 80
ENC, DEC, JOIN, V = 512, 512, 512, 500
U = 32
U1 = U + 1
NEG = -1e30


def _joint_body(x_ref, y_ref, we_ref, be_ref, emb_ref, wep_ref, bep_ref,
                wdp_ref, bdp_ref, wo_ref, bo_ref, blank_ref, ylp_ref):
    xb = x_ref[0]
    enc = jnp.tanh(jnp.dot(xb, we_ref[...]) + be_ref[...])
    ep = jnp.dot(enc, wep_ref[...]) + bep_ref[...]

    ids = jnp.concatenate(
        [jnp.zeros((1, 1), jnp.int32), y_ref[0]], axis=0)
    vlane = lax.broadcasted_iota(jnp.int32, (U1, V), 1)
    oh = (vlane == ids).astype(jnp.float32)
    dec = jnp.tanh(jnp.dot(oh, emb_ref[...]))
    dp = jnp.dot(dec, wdp_ref[...]) + bdp_ref[...]

    lane_u1 = lax.broadcasted_iota(jnp.int32, (T, U1), 1)
    lane_u = lax.broadcasted_iota(jnp.int32, (T, U), 1)
    blank_acc = jnp.zeros((T, U1), jnp.float32)
    y_acc = jnp.zeros((T, U), jnp.float32)
    for u in range(U1):
        jt = jnp.tanh(ep + dp[u:u + 1, :])
        lg = jnp.dot(jt, wo_ref[...]) + bo_ref[...]
        m = jnp.max(lg, axis=1, keepdims=True)
        lse = m + jnp.log(jnp.sum(jnp.exp(lg - m), axis=1, keepdims=True))
        bl = lg[:, 0:1] - lse
        blank_acc = jnp.where(lane_u1 == u, bl, blank_acc)
        if u < U:
            yv = jnp.sum(lg * oh[u + 1:u + 2, :], axis=1, keepdims=True) - lse
            y_acc = jnp.where(lane_u == u, yv, y_acc)
    blank_ref[0] = blank_acc
    ylp_ref[0] = y_acc


def _logaddexp(a, b):
    mx = jnp.maximum(a, b)
    mn = jnp.minimum(a, b)
    return mx + jnp.log(1.0 + jnp.exp(mn - mx))


def _dp_body(blank_ref, ylp_ref, tl_ref, ul_ref, out_ref):
    lane = lax.broadcasted_iota(jnp.int32, (B, U1), 1)
    tl = tl_ref[...]
    ul = ul_ref[...]
    uoh = lane == ul
    init_b = jnp.where(lane == 0, 0.0, NEG)

    def shift(v, s, fill):
        pad = jnp.full((B, s), fill, jnp.float32)
        return jnp.concatenate([pad, v[:, :U1 - s]], axis=1)

    def body(t, carry):
        alpha, blank_prev, acc = carry
        yrow = jnp.concatenate(
            [ylp_ref[b, pl.ds(t, 1), :] for b in range(B)], axis=0)
        w = jnp.concatenate([jnp.zeros((B, 1), jnp.float32), yrow], axis=1)
        bvec = jnp.where(t == 0, init_b, alpha + blank_prev)
        W = w
        for s in (1, 2, 4, 8, 16, 32):
            W = W + shift(W, s, 0.0)
        c = bvec - W
        for s in (1, 2, 4, 8, 16, 32):
            c = _logaddexp(c, shift(c, s, NEG))
        alpha_new = c + W
        blank_t = jnp.concatenate(
            [blank_ref[b, pl.ds(t, 1), :] for b in range(B)], axis=0)
        hit = (tl - 1) == t
        contrib = jnp.where(jnp.logical_and(hit, uoh),
                            alpha_new + blank_t, 0.0)
        return alpha_new, blank_t, acc + contrib

    zero = jnp.zeros((B, U1), jnp.float32)
    _, _, acc = lax.fori_loop(0, T, body, (zero, zero, zero))
    out_ref[...] = (-jnp.sum(acc)).reshape(1, 1)


@jax.jit
def kernel(x, x_lens, y_padded, y_lens, W_enc, b_enc, embed, W_encp, b_encp,
           W_decp, b_decp, W_out, b_out):
    y3 = y_padded.astype(jnp.int32).reshape(B, U, 1)
    blank_lp, y_lp = pl.pallas_call(
        _joint_body,
        grid=(B,),
        in_specs=[
            pl.BlockSpec((1, T, F), lambda b: (b, 0, 0)),
            pl.BlockSpec((1, U, 1), lambda b: (b, 0, 0)),
            pl.BlockSpec((F, ENC), lambda b: (0, 0)),
            pl.BlockSpec((1, ENC), lambda b: (0, 0)),
            pl.BlockSpec((V, DEC), lambda b: (0, 0)),
            pl.BlockSpec((ENC, JOIN), lambda b: (0, 0)),
            pl.BlockSpec((1, JOIN), lambda b: (0, 0)),
            pl.BlockSpec((DEC, JOIN), lambda b: (0, 0)),
            pl.BlockSpec((1, JOIN), lambda b: (0, 0)),
            pl.BlockSpec((JOIN, V), lambda b: (0, 0)),
            pl.BlockSpec((1, V), lambda b: (0, 0)),
        ],
        out_specs=[
            pl.BlockSpec((1, T, U1), lambda b: (b, 0, 0)),
            pl.BlockSpec((1, T, U), lambda b: (b, 0, 0)),
        ],
        out_shape=[
            jax.ShapeDtypeStruct((B, T, U1), jnp.float32),
            jax.ShapeDtypeStruct((B, T, U), jnp.float32),
        ],
    )(x, y3, W_enc, b_enc.reshape(1, ENC), embed, W_encp,
      b_encp.reshape(1, JOIN), W_decp, b_decp.reshape(1, JOIN), W_out,
      b_out.reshape(1, V))

    out = pl.pallas_call(
        _dp_body,
        out_shape=jax.ShapeDtypeStruct((1, 1), jnp.float32),
    )(blank_lp, y_lp,
      x_lens.astype(jnp.int32).reshape(B, 1),
      y_lens.astype(jnp.int32).reshape(B, 1))
    return out[0, 0]
